# baseline (speedup 1.0000x reference)
import functools

import jax
import jax.numpy as jnp
from jax import lax
from jax.experimental import pallas as pl
from jax.experimental.pallas import tpu as pltpu
from jax.experimental.pallas import tpu_sc as plsc

NC = 2
NS = 16
L = 16
NW = NC * NS

B = 8
N = 2048
M = 2048
BSC = 1
NT = 1024
NPT = N // NT
KSKIP = 2
F0 = KSKIP * NT
CHUNK = F0 // NW
S = 256
NQ = M // S
TT = S // L
PG = 2
PGROUPS = CHUNK // (PG * L)
KA = 8


def _prep_rep_body(vt_ref, rep_ref):
  y = vt_ref[0]
  yb = y.astype(jnp.bfloat16).astype(jnp.float32)
  m = yb * (-2.0)
  y2 = jnp.sum(y * y, axis=0, keepdims=True)
  rows4 = jnp.concatenate([m, y2], axis=0)
  rep16 = jnp.broadcast_to(rows4[:, None, :], (4, L, M)).reshape(4 * L, M)
  rep_ref[...] = jnp.transpose(rep16)


@jax.jit
def _prep_rep(v_t):
  return pl.pallas_call(
      _prep_rep_body,
      grid=(BSC,),
      in_specs=[pl.BlockSpec((1, 3, M), lambda i: (i, 0, 0))],
      out_specs=pl.BlockSpec((M, 4 * L), lambda i: (0, 0)),
      out_shape=jax.ShapeDtypeStruct((M, 4 * L), jnp.float32),
  )(v_t)


def _round_bf16(x):
  u = lax.bitcast_convert_type(x, jnp.uint32)
  r = u + jnp.uint32(0x7FFF) + ((u >> jnp.uint32(16)) & jnp.uint32(1))
  r = r & jnp.uint32(0xFFFF0000)
  return lax.bitcast_convert_type(r, jnp.float32)


def _chamfer_body(rep_hbm, vp_hbm, out_hbm,
                  buf0, buf1, vp, minv, psum, sem0, sem1):
  wid = lax.axis_index("s") * NC + lax.axis_index("c")
  for c in range(3):
    pltpu.sync_copy(vp_hbm.at[c, pl.ds(wid * CHUNK, CHUNK)], vp.at[c])

  bufs = (buf0, buf1)
  sems = (sem0, sem1)
  big = jnp.full((L,), 3.0e38, dtype=jnp.float32)
  zero = jnp.zeros((L,), dtype=jnp.float32)
  psum[...] = zero
  for r in range(CHUNK // L):
    minv[pl.ds(r * L, L)] = big

  q0 = lax.rem(wid, NQ)

  def _off(k):
    return lax.rem(q0 + k, NQ) * S

  pltpu.async_copy(rep_hbm.at[pl.ds(_off(0), S)], buf0, sem0)

  def _process(k, buf, sem, nbuf, nsem):
    @pl.when(k + 1 < NQ)
    def _prefetch():
      pltpu.async_copy(rep_hbm.at[pl.ds(_off(k + 1), S)], nbuf, nsem)

    pltpu.make_async_copy(rep_hbm.at[pl.ds(0, S)], buf, sem).wait()

    @pl.loop(0, PGROUPS)
    def _grp(g, _buf=buf):
      gbase = g * (PG * L)
      px = []
      py = []
      pz = []
      for p in range(PG):
        s = pl.ds(gbase + p * L, L)
        px.append(_round_bf16(vp[0, s]))
        py.append(_round_bf16(vp[1, s]))
        pz.append(_round_bf16(vp[2, s]))

      def _tt(tt, accs):
        accs = list(accs)
        for j in range(L):
          row = tt * L + j
          g0 = _buf[row, pl.ds(0, L)]
          g1 = _buf[row, pl.ds(L, L)]
          g2 = _buf[row, pl.ds(2 * L, L)]
          gs = _buf[row, pl.ds(3 * L, L)]
          for p in range(PG):
            d = px[p] * g0 + py[p] * g1
            d = d + pz[p] * g2
            d = d + gs
            accs[p] = jnp.minimum(accs[p], d)
        return tuple(accs)

      accs = lax.fori_loop(0, TT, _tt, tuple(big for _ in range(PG)))

      for p in range(PG):
        s = pl.ds(gbase + p * L, L)
        minv[s] = jnp.minimum(minv[s], accs[p])

  @pl.loop(0, NQ)
  def _k(k):
    @pl.when(lax.rem(k, 2) == 0)
    def _even():
      _process(k, buf0, sem0, buf1, sem1)

    @pl.when(lax.rem(k, 2) == 1)
    def _odd():
      _process(k, buf1, sem1, buf0, sem0)

  for r in range(CHUNK // L):
    s = pl.ds(r * L, L)
    a = vp[0, s]
    c = vp[1, s]
    d = vp[2, s]
    x2 = a * a + c * c
    x2 = x2 + d * d
    psum[...] = psum[...] + jnp.maximum(minv[s] + x2, zero)

  pltpu.sync_copy(psum, out_hbm.at[wid])


@jax.jit
def _chamfer_sc(rep, vp_soa):
  mesh = plsc.VectorSubcoreMesh(
      core_axis_name="c", subcore_axis_name="s", num_cores=NC, num_subcores=NS)
  f = pl.kernel(
      _chamfer_body,
      out_type=jax.ShapeDtypeStruct((NW, L), jnp.float32),
      mesh=mesh,
      scratch_types=[
          pltpu.VMEM((S, 4 * L), jnp.float32),
          pltpu.VMEM((S, 4 * L), jnp.float32),
          pltpu.VMEM((3, CHUNK), jnp.float32),
          pltpu.VMEM((CHUNK,), jnp.float32),
          pltpu.VMEM((L,), jnp.float32),
          pltpu.SemaphoreType.DMA,
          pltpu.SemaphoreType.DMA,
      ],
  )
  return f(rep, vp_soa)


def _tc_chamfer_body(vp_ref, vt_ref, out_ref, yaug_s):
  b = pl.program_id(0)
  i = pl.program_id(1)

  out_ref[b, i] = 0.0

  @pl.when(jnp.logical_and(i == 0, jnp.logical_or(b > 0, KSKIP < NPT)))
  def _build_yaug():
    y = vt_ref[0]
    yb = y.astype(jnp.bfloat16).astype(jnp.float32)
    m = yb * (-2.0)
    y2 = jnp.sum(y * y, axis=0, keepdims=True)
    y2hi = y2.astype(jnp.bfloat16).astype(jnp.float32)
    y2lo = (y2 - y2hi).astype(jnp.bfloat16).astype(jnp.float32)
    zero = jnp.zeros((KA - 5, M), jnp.float32)
    yaug_s[...] = jnp.concatenate([m, y2hi, y2lo, zero], axis=0)

  @pl.when(jnp.logical_or(b > 0, i >= KSKIP))
  def _main():
    x = vp_ref[0]
    x2 = jnp.sum(x * x, axis=0)
    xb = x.astype(jnp.bfloat16)
    ones = jnp.ones((2, NT), jnp.bfloat16)
    zero = jnp.zeros((KA - 5, NT), jnp.bfloat16)
    xaug = jnp.concatenate([xb, ones, zero], axis=0)
    yb = yaug_s[...].astype(jnp.bfloat16)
    t = lax.dot_general(xaug, yb, (((0,), (0,)), ((), ())),
                        preferred_element_type=jnp.float32)
    mn = jnp.min(t, axis=1)
    out_ref[b, i] = jnp.sum(jnp.maximum(mn + x2, 0.0))


@jax.jit
def _chamfer_tc(vp_t, v_t):
  return pl.pallas_call(
      _tc_chamfer_body,
      grid=(B, NPT),
      in_specs=[
          pl.BlockSpec((1, 3, NT), lambda b, i: (b, 0, i)),
          pl.BlockSpec((1, 3, M), lambda b, i: (b, 0, 0)),
      ],
      out_specs=pl.BlockSpec((B, NPT), lambda b, i: (0, 0),
                             memory_space=pltpu.SMEM),
      out_shape=jax.ShapeDtypeStruct((B, NPT), jnp.float32),
      scratch_shapes=[pltpu.VMEM((KA, M), jnp.float32)],
  )(vp_t, v_t)


def kernel(v, v_pred):
  v_t = jnp.transpose(v, (0, 2, 1))
  vp_t = jnp.transpose(v_pred, (0, 2, 1))
  rep = _prep_rep(v_t)
  sc_part = _chamfer_sc(rep, vp_t[0])
  tc_part = _chamfer_tc(vp_t, v_t)
  return (jnp.sum(sc_part) + jnp.sum(tc_part)) / jnp.float32(B * N)


# speedup vs baseline: 1.0287x; 1.0287x over previous
import jax
import jax.numpy as jnp
from jax import lax
from jax.experimental import pallas as pl
from jax.experimental.pallas import tpu as pltpu
from jax.experimental.pallas import tpu_sc as plsc

NC = 2
NS = 16
L = 16
NW = NC * NS

B = 8
N = 2048
M = 2048
BSC = 1
NT = 1024
NPT = N // NT
KSKIP = 2
F0 = KSKIP * NT
CHUNK = F0 // NW
S = 256
NQ = M // S
TT = S // L
PG = 2
PGROUPS = CHUNK // (PG * L)
KA = 8


def _prep_rep_body(vt_ref, rep_ref):
  y = vt_ref[0]
  yb = y.astype(jnp.bfloat16).astype(jnp.float32)
  m = yb * (-2.0)
  y2 = jnp.sum(y * y, axis=0, keepdims=True)
  rows4 = jnp.concatenate([m, y2], axis=0)
  rep16 = jnp.broadcast_to(rows4[:, None, :], (4, L, M)).reshape(4 * L, M)
  rep_ref[...] = jnp.transpose(rep16)


@jax.jit
def _prep_rep(v_t):
  return pl.pallas_call(
      _prep_rep_body,
      grid=(BSC,),
      in_specs=[pl.BlockSpec((1, 3, M), lambda i: (i, 0, 0))],
      out_specs=pl.BlockSpec((M, 4 * L), lambda i: (0, 0)),
      out_shape=jax.ShapeDtypeStruct((M, 4 * L), jnp.float32),
  )(v_t)


def _round_bf16(x):
  u = lax.bitcast_convert_type(x, jnp.uint32)
  r = u + jnp.uint32(0x7FFF) + ((u >> jnp.uint32(16)) & jnp.uint32(1))
  r = r & jnp.uint32(0xFFFF0000)
  return lax.bitcast_convert_type(r, jnp.float32)


def _chamfer_body(rep_hbm, vp_hbm, out_hbm,
                  buf0, buf1, vp, minv, psum, sem0, sem1):
  wid = lax.axis_index("s") * NC + lax.axis_index("c")
  for c in range(3):
    pltpu.sync_copy(vp_hbm.at[c, pl.ds(wid * CHUNK, CHUNK)], vp.at[c])

  bufs = (buf0, buf1)
  sems = (sem0, sem1)
  big = jnp.full((L,), 3.0e38, dtype=jnp.float32)
  zero = jnp.zeros((L,), dtype=jnp.float32)
  psum[...] = zero
  for r in range(CHUNK // L):
    minv[pl.ds(r * L, L)] = big

  q0 = lax.rem(wid, NQ)

  def _off(k):
    return lax.rem(q0 + k, NQ) * S

  pltpu.async_copy(rep_hbm.at[pl.ds(_off(0), S)], buf0, sem0)

  def _process(k, buf, sem, nbuf, nsem):
    @pl.when(k + 1 < NQ)
    def _prefetch():
      pltpu.async_copy(rep_hbm.at[pl.ds(_off(k + 1), S)], nbuf, nsem)

    pltpu.make_async_copy(rep_hbm.at[pl.ds(0, S)], buf, sem).wait()

    @pl.loop(0, PGROUPS)
    def _grp(g, _buf=buf):
      gbase = g * (PG * L)
      px = []
      py = []
      pz = []
      for p in range(PG):
        s = pl.ds(gbase + p * L, L)
        px.append(_round_bf16(vp[0, s]))
        py.append(_round_bf16(vp[1, s]))
        pz.append(_round_bf16(vp[2, s]))

      def _tt(tt, accs):
        accs = list(accs)
        for j in range(L):
          row = tt * L + j
          g0 = _buf[row, pl.ds(0, L)]
          g1 = _buf[row, pl.ds(L, L)]
          g2 = _buf[row, pl.ds(2 * L, L)]
          gs = _buf[row, pl.ds(3 * L, L)]
          for p in range(PG):
            d = px[p] * g0 + py[p] * g1
            d = d + pz[p] * g2
            d = d + gs
            accs[p] = jnp.minimum(accs[p], d)
        return tuple(accs)

      accs = lax.fori_loop(0, TT, _tt, tuple(big for _ in range(PG)))

      for p in range(PG):
        s = pl.ds(gbase + p * L, L)
        minv[s] = jnp.minimum(minv[s], accs[p])

  @pl.loop(0, NQ)
  def _k(k):
    @pl.when(lax.rem(k, 2) == 0)
    def _even():
      _process(k, buf0, sem0, buf1, sem1)

    @pl.when(lax.rem(k, 2) == 1)
    def _odd():
      _process(k, buf1, sem1, buf0, sem0)

  for r in range(CHUNK // L):
    s = pl.ds(r * L, L)
    a = vp[0, s]
    c = vp[1, s]
    d = vp[2, s]
    x2 = a * a + c * c
    x2 = x2 + d * d
    psum[...] = psum[...] + jnp.maximum(minv[s] + x2, zero)

  pltpu.sync_copy(psum, out_hbm.at[wid])


@jax.jit
def _chamfer_sc(rep, vp_soa):
  mesh = plsc.VectorSubcoreMesh(
      core_axis_name="c", subcore_axis_name="s", num_cores=NC, num_subcores=NS)
  f = pl.kernel(
      _chamfer_body,
      out_type=jax.ShapeDtypeStruct((NW, L), jnp.float32),
      mesh=mesh,
      scratch_types=[
          pltpu.VMEM((S, 4 * L), jnp.float32),
          pltpu.VMEM((S, 4 * L), jnp.float32),
          pltpu.VMEM((3, CHUNK), jnp.float32),
          pltpu.VMEM((CHUNK,), jnp.float32),
          pltpu.VMEM((L,), jnp.float32),
          pltpu.SemaphoreType.DMA,
          pltpu.SemaphoreType.DMA,
      ],
  )
  return f(rep, vp_soa)


def _tc_chamfer_body(vp_ref, vt_ref, out_ref, yaug_s):
  b = pl.program_id(0)
  i = pl.program_id(1)

  @pl.when(jnp.logical_and(b == 0, i == 0))
  def _init():
    out_ref[0, 0] = 0.0

  @pl.when(jnp.logical_and(i == 0, jnp.logical_or(b > 0, KSKIP < NPT)))
  def _build_yaug():
    y = vt_ref[0]
    yb = y.astype(jnp.bfloat16).astype(jnp.float32)
    m = yb * (-2.0)
    y2 = jnp.sum(y * y, axis=0, keepdims=True)
    y2hi = y2.astype(jnp.bfloat16).astype(jnp.float32)
    y2lo = (y2 - y2hi).astype(jnp.bfloat16).astype(jnp.float32)
    zero = jnp.zeros((KA - 5, M), jnp.float32)
    yaug_s[...] = jnp.concatenate([m, y2hi, y2lo, zero], axis=0)

  @pl.when(jnp.logical_or(b > 0, i >= KSKIP))
  def _main():
    x = vp_ref[0]
    x2 = jnp.sum(x * x, axis=0)
    xb = x.astype(jnp.bfloat16)
    ones = jnp.ones((2, NT), jnp.bfloat16)
    zero = jnp.zeros((KA - 5, NT), jnp.bfloat16)
    xaug = jnp.concatenate([xb, ones, zero], axis=0)
    yb = yaug_s[...].astype(jnp.bfloat16)
    t = lax.dot_general(xaug, yb, (((0,), (0,)), ((), ())),
                        preferred_element_type=jnp.float32)
    mn = jnp.min(t, axis=1)
    out_ref[0, 0] += jnp.sum(jnp.maximum(mn + x2, 0.0))


@jax.jit
def _chamfer_tc(vp_t, v_t):
  return pl.pallas_call(
      _tc_chamfer_body,
      grid=(B, NPT),
      in_specs=[
          pl.BlockSpec((1, 3, NT), lambda b, i: (b, 0, i)),
          pl.BlockSpec((1, 3, M), lambda b, i: (b, 0, 0)),
      ],
      out_specs=pl.BlockSpec((1, 1), lambda b, i: (0, 0),
                             memory_space=pltpu.SMEM),
      out_shape=jax.ShapeDtypeStruct((1, 1), jnp.float32),
      scratch_shapes=[pltpu.VMEM((KA, M), jnp.float32)],
  )(vp_t, v_t)


def kernel(v, v_pred):
  v_t = jnp.transpose(v, (0, 2, 1))
  vp_t = jnp.transpose(v_pred, (0, 2, 1))
  rep = _prep_rep(v_t)
  sc_part = _chamfer_sc(rep, vp_t[0])
  tc_sum = _chamfer_tc(vp_t, v_t)
  return (jnp.sum(sc_part) + tc_sum[0, 0]) / jnp.float32(B * N)


# speedup vs baseline: 1.0389x; 1.0099x over previous
import jax
import jax.numpy as jnp
from jax import lax
from jax.experimental import pallas as pl
from jax.experimental.pallas import tpu as pltpu
from jax.experimental.pallas import tpu_sc as plsc

NC = 2
NS = 16
L = 16
NW = NC * NS

B = 8
N = 2048
M = 2048
BSC = 1
NT = 2048
NPT = N // NT
KSKIP = 1
F0 = KSKIP * NT
CHUNK = F0 // NW
S = 256
NQ = M // S
TT = S // L
PG = 2
PGROUPS = CHUNK // (PG * L)
KA = 8


def _prep_rep_body(vt_ref, rep_ref):
  y = vt_ref[0]
  yb = y.astype(jnp.bfloat16).astype(jnp.float32)
  m = yb * (-2.0)
  y2 = jnp.sum(y * y, axis=0, keepdims=True)
  rows4 = jnp.concatenate([m, y2], axis=0)
  rep16 = jnp.broadcast_to(rows4[:, None, :], (4, L, M)).reshape(4 * L, M)
  rep_ref[...] = jnp.transpose(rep16)


@jax.jit
def _prep_rep(v_t):
  return pl.pallas_call(
      _prep_rep_body,
      grid=(BSC,),
      in_specs=[pl.BlockSpec((1, 3, M), lambda i: (i, 0, 0))],
      out_specs=pl.BlockSpec((M, 4 * L), lambda i: (0, 0)),
      out_shape=jax.ShapeDtypeStruct((M, 4 * L), jnp.float32),
  )(v_t)


def _round_bf16(x):
  u = lax.bitcast_convert_type(x, jnp.uint32)
  r = u + jnp.uint32(0x7FFF) + ((u >> jnp.uint32(16)) & jnp.uint32(1))
  r = r & jnp.uint32(0xFFFF0000)
  return lax.bitcast_convert_type(r, jnp.float32)


def _chamfer_body(rep_hbm, vp_hbm, out_hbm,
                  buf0, buf1, vp, minv, psum, sem0, sem1):
  wid = lax.axis_index("s") * NC + lax.axis_index("c")
  for c in range(3):
    pltpu.sync_copy(vp_hbm.at[c, pl.ds(wid * CHUNK, CHUNK)], vp.at[c])

  bufs = (buf0, buf1)
  sems = (sem0, sem1)
  big = jnp.full((L,), 3.0e38, dtype=jnp.float32)
  zero = jnp.zeros((L,), dtype=jnp.float32)
  psum[...] = zero
  for r in range(CHUNK // L):
    minv[pl.ds(r * L, L)] = big

  q0 = lax.rem(wid, NQ)

  def _off(k):
    return lax.rem(q0 + k, NQ) * S

  pltpu.async_copy(rep_hbm.at[pl.ds(_off(0), S)], buf0, sem0)

  def _process(k, buf, sem, nbuf, nsem):
    @pl.when(k + 1 < NQ)
    def _prefetch():
      pltpu.async_copy(rep_hbm.at[pl.ds(_off(k + 1), S)], nbuf, nsem)

    pltpu.make_async_copy(rep_hbm.at[pl.ds(0, S)], buf, sem).wait()

    @pl.loop(0, PGROUPS)
    def _grp(g, _buf=buf):
      gbase = g * (PG * L)
      px = []
      py = []
      pz = []
      for p in range(PG):
        s = pl.ds(gbase + p * L, L)
        px.append(_round_bf16(vp[0, s]))
        py.append(_round_bf16(vp[1, s]))
        pz.append(_round_bf16(vp[2, s]))

      def _tt(tt, accs):
        accs = list(accs)
        for j in range(L):
          row = tt * L + j
          g0 = _buf[row, pl.ds(0, L)]
          g1 = _buf[row, pl.ds(L, L)]
          g2 = _buf[row, pl.ds(2 * L, L)]
          gs = _buf[row, pl.ds(3 * L, L)]
          for p in range(PG):
            d = px[p] * g0 + py[p] * g1
            d = d + pz[p] * g2
            d = d + gs
            accs[p] = jnp.minimum(accs[p], d)
        return tuple(accs)

      accs = lax.fori_loop(0, TT, _tt, tuple(big for _ in range(PG)))

      for p in range(PG):
        s = pl.ds(gbase + p * L, L)
        minv[s] = jnp.minimum(minv[s], accs[p])

  @pl.loop(0, NQ)
  def _k(k):
    @pl.when(lax.rem(k, 2) == 0)
    def _even():
      _process(k, buf0, sem0, buf1, sem1)

    @pl.when(lax.rem(k, 2) == 1)
    def _odd():
      _process(k, buf1, sem1, buf0, sem0)

  for r in range(CHUNK // L):
    s = pl.ds(r * L, L)
    a = vp[0, s]
    c = vp[1, s]
    d = vp[2, s]
    x2 = a * a + c * c
    x2 = x2 + d * d
    psum[...] = psum[...] + jnp.maximum(minv[s] + x2, zero)

  pltpu.sync_copy(psum, out_hbm.at[wid])


@jax.jit
def _chamfer_sc(rep, vp_soa):
  mesh = plsc.VectorSubcoreMesh(
      core_axis_name="c", subcore_axis_name="s", num_cores=NC, num_subcores=NS)
  f = pl.kernel(
      _chamfer_body,
      out_type=jax.ShapeDtypeStruct((NW, L), jnp.float32),
      mesh=mesh,
      scratch_types=[
          pltpu.VMEM((S, 4 * L), jnp.float32),
          pltpu.VMEM((S, 4 * L), jnp.float32),
          pltpu.VMEM((3, CHUNK), jnp.float32),
          pltpu.VMEM((CHUNK,), jnp.float32),
          pltpu.VMEM((L,), jnp.float32),
          pltpu.SemaphoreType.DMA,
          pltpu.SemaphoreType.DMA,
      ],
  )
  return f(rep, vp_soa)


def _tc_chamfer_body(vp_ref, vt_ref, out_ref, yaug_s):
  b = pl.program_id(0)
  i = pl.program_id(1)

  @pl.when(jnp.logical_and(b == 0, i == 0))
  def _init():
    out_ref[0, 0] = 0.0

  @pl.when(jnp.logical_and(i == 0, jnp.logical_or(b > 0, KSKIP < NPT)))
  def _build_yaug():
    y = vt_ref[0]
    yb = y.astype(jnp.bfloat16).astype(jnp.float32)
    m = yb * (-2.0)
    y2 = jnp.sum(y * y, axis=0, keepdims=True)
    y2hi = y2.astype(jnp.bfloat16).astype(jnp.float32)
    y2lo = (y2 - y2hi).astype(jnp.bfloat16).astype(jnp.float32)
    zero = jnp.zeros((KA - 5, M), jnp.float32)
    yaug_s[...] = jnp.concatenate([m, y2hi, y2lo, zero], axis=0)

  @pl.when(jnp.logical_or(b > 0, i >= KSKIP))
  def _main():
    x = vp_ref[0]
    x2 = jnp.sum(x * x, axis=0)
    xb = x.astype(jnp.bfloat16)
    ones = jnp.ones((2, NT), jnp.bfloat16)
    zero = jnp.zeros((KA - 5, NT), jnp.bfloat16)
    xaug = jnp.concatenate([xb, ones, zero], axis=0)
    yb = yaug_s[...].astype(jnp.bfloat16)
    t = lax.dot_general(xaug, yb, (((0,), (0,)), ((), ())),
                        preferred_element_type=jnp.float32)
    mn = jnp.min(t, axis=1)
    out_ref[0, 0] += jnp.sum(jnp.maximum(mn + x2, 0.0))


@jax.jit
def _chamfer_tc(vp_t, v_t):
  return pl.pallas_call(
      _tc_chamfer_body,
      grid=(B, NPT),
      in_specs=[
          pl.BlockSpec((1, 3, NT), lambda b, i: (b, 0, i)),
          pl.BlockSpec((1, 3, M), lambda b, i: (b, 0, 0)),
      ],
      out_specs=pl.BlockSpec((1, 1), lambda b, i: (0, 0),
                             memory_space=pltpu.SMEM),
      out_shape=jax.ShapeDtypeStruct((1, 1), jnp.float32),
      scratch_shapes=[pltpu.VMEM((KA, M), jnp.float32)],
  )(vp_t, v_t)


def kernel(v, v_pred):
  v_t = jnp.transpose(v, (0, 2, 1))
  vp_t = jnp.transpose(v_pred, (0, 2, 1))
  rep = _prep_rep(v_t)
  sc_part = _chamfer_sc(rep, vp_t[0])
  tc_sum = _chamfer_tc(vp_t, v_t)
  return (jnp.sum(sc_part) + tc_sum[0, 0]) / jnp.float32(B * N)
